# baseline (speedup 1.0000x reference)
import functools

import jax
import jax.numpy as jnp
from jax import lax
from jax.experimental import pallas as pl
from jax.experimental.pallas import tpu as pltpu
from jax.experimental.pallas import tpu_sc as plsc

N = 1024
E = 16384
EDGE_DIM = 16
MAX_PATH = 5

_NC, _NS = 2, 16
_NW = _NC * _NS
_RPW = N // _NW
_ROW_W = N * MAX_PATH
_TBL_W = MAX_PATH * E


def _tc_table_body(evp_ref, eat_ref, o_ref):
    o_ref[...] = lax.dot_general(
        evp_ref[...], eat_ref[...], (((1,), (0,)), ((), ())),
        preferred_element_type=jnp.float32) * (1.0 / MAX_PATH)


def _make_table(edge_attr, edge_vector):
    evp = jnp.zeros((8, EDGE_DIM), jnp.float32).at[:MAX_PATH].set(edge_vector)
    eat = edge_attr.T
    out = pl.pallas_call(
        _tc_table_body,
        out_shape=jax.ShapeDtypeStruct((8, E), jnp.float32),
    )(evp, eat)
    pairs = out[:MAX_PATH].astype(jnp.bfloat16).reshape(MAX_PATH, E // 2, 2)
    return lax.bitcast_convert_type(pairs, jnp.int32).reshape(-1)


_G = 8
_NG = _RPW // _G
_GW = _G * N


_TBLP_W = _TBL_W // 2
_EH = E // 2


def _sc_body(table_hbm, paths_hbm, out_hbm, tbl_v,
             a0, a1, a2, a3, a4, b0, b1, b2, b3, obuf,
             sem_t, sem_i, sem_o):
    wid = lax.axis_index("s") * _NC + lax.axis_index("c")
    base = wid * _RPW
    sets = ((a0, a1, a2, a3, a4), (b0, b1, b2, b3, a4))

    def start_in(g, planes):
        r = base + g * _G
        bufs = sets[g & 1]
        for d in planes:
            pltpu.async_copy(
                paths_hbm.at[d, pl.ds(r, _G)], bufs[d], sem_i)

    def drain_in():
        for d in range(MAX_PATH):
            pltpu.make_async_copy(
                paths_hbm.at[0, pl.ds(0, _G)], a0, sem_i).wait()

    def out_slice(g):
        return out_hbm.at[pl.ds(base + g * _G, _G)]

    pltpu.async_copy(table_hbm.at[pl.ds(0, _TBLP_W)], tbl_v, sem_t)
    start_in(0, range(MAX_PATH))
    pltpu.make_async_copy(
        table_hbm.at[pl.ds(0, _TBLP_W)], tbl_v, sem_t).wait()

    for g in range(_NG):
        drain_in()
        if g + 1 < _NG:
            start_in(g + 1, range(4))
        if g > 0:
            pltpu.make_async_copy(out_slice(0), obuf, sem_o).wait()
        pbufs = sets[g & 1]

        @plsc.parallel_loop(0, _GW // 16, 1, unroll=8)
        def chunk(c):
            q = c * 16
            i = q >> 10
            j = q & (N - 1)
            acc = jnp.zeros((16,), jnp.float32)
            for d in range(MAX_PATH):
                p = pbufs[d][i, pl.ds(j, 16)]
                w = plsc.load_gather(
                    tbl_v.at[pl.ds(d * _EH, _EH)], [p >> 1])
                bits = (w >> ((p & 1) << 4)) << 16
                acc = acc + plsc.bitcast(bits, jnp.float32)
            obuf[i, pl.ds(j, 16)] = acc

        pltpu.async_copy(obuf, out_slice(g), sem_o)
        if g + 1 < _NG:
            start_in(g + 1, [4])
    pltpu.make_async_copy(out_slice(0), obuf, sem_o).wait()


@functools.cache
def _sc_call():
    return functools.partial(
        pl.kernel,
        out_type=jax.ShapeDtypeStruct((N, N), jnp.float32),
        mesh=plsc.VectorSubcoreMesh(
            core_axis_name="c", subcore_axis_name="s",
            num_cores=_NC, num_subcores=_NS),
        compiler_params=pltpu.CompilerParams(needs_layout_passes=False),
        scratch_types=[
            pltpu.VMEM((_TBLP_W,), jnp.int32),
        ] + [pltpu.VMEM((_G, N), jnp.int32) for _ in range(9)] + [
            pltpu.VMEM((_G, N), jnp.float32),
            pltpu.SemaphoreType.DMA,
            pltpu.SemaphoreType.DMA,
            pltpu.SemaphoreType.DMA,
        ],
    )(_sc_body)


def kernel(x, edge_attr, edge_paths, edge_vector):
    del x
    paths = jnp.transpose(edge_paths.astype(jnp.int32), (2, 0, 1))
    table = _make_table(edge_attr.astype(jnp.float32),
                        edge_vector.astype(jnp.float32))
    return _sc_call()(table, paths)


# speedup vs baseline: 1.7743x; 1.7743x over previous
import functools

import jax
import jax.numpy as jnp
from jax import lax
from jax.experimental import pallas as pl
from jax.experimental.pallas import tpu as pltpu
from jax.experimental.pallas import tpu_sc as plsc

N = 1024
E = 16384
EDGE_DIM = 16
MAX_PATH = 5

_NC, _NS = 2, 16
_NW = _NC * _NS
_RPW = N // _NW
_ROW_W = N * MAX_PATH
_TBL_W = MAX_PATH * E


def _tc_table_body(evp_ref, eat_ref, o_ref):
    o_ref[...] = lax.dot_general(
        evp_ref[...], eat_ref[...], (((1,), (0,)), ((), ())),
        preferred_element_type=jnp.float32) * (1.0 / MAX_PATH)


def _make_table(edge_attr, edge_vector):
    evp = jnp.zeros((8, EDGE_DIM), jnp.float32).at[:MAX_PATH].set(edge_vector)
    eat = edge_attr.T
    out = pl.pallas_call(
        _tc_table_body,
        out_shape=jax.ShapeDtypeStruct((8, E), jnp.float32),
    )(evp, eat)
    return out.reshape(-1)


_G = 8
_NG = _RPW // _G
_GW = _G * N


def _sc_body(table_hbm, paths_hbm, out_hbm, tbl_v,
             pb0, pb1, pb2, pb3, pb4, obuf, sem_t, sem_i, sem_o):
    wid = lax.axis_index("s") * _NC + lax.axis_index("c")
    base = wid * _RPW
    pbufs = (pb0, pb1, pb2, pb3, pb4)

    def start_in(g, planes):
        r = base + g * _G
        for d in planes:
            pltpu.async_copy(
                paths_hbm.at[d, pl.ds(r, _G)], pbufs[d], sem_i)

    def drain_in():
        for d in range(MAX_PATH):
            pltpu.make_async_copy(
                paths_hbm.at[0, pl.ds(0, _G)], pbufs[d], sem_i).wait()

    def out_slice(g):
        return out_hbm.at[pl.ds(base + g * _G, _G)]

    pltpu.async_copy(table_hbm.at[pl.ds(0, _TBL_W)], tbl_v, sem_t)
    start_in(0, range(MAX_PATH))
    pltpu.make_async_copy(
        table_hbm.at[pl.ds(0, _TBL_W)], tbl_v, sem_t).wait()

    for g in range(_NG):
        drain_in()

        if g > 0:
            pltpu.make_async_copy(out_slice(0), obuf, sem_o).wait()

        @plsc.parallel_loop(0, _GW // 16, 1, unroll=8)
        def chunk(c):
            q = c * 16
            i = q >> 10
            j = q & (N - 1)
            acc = jnp.zeros((16,), jnp.float32)
            for d in range(MAX_PATH):
                p = pbufs[d][i, pl.ds(j, 16)]
                acc = acc + plsc.load_gather(
                    tbl_v.at[pl.ds(d * E, E)], [p])
            obuf[i, pl.ds(j, 16)] = acc

        pltpu.async_copy(obuf, out_slice(g), sem_o)
        if g + 1 < _NG:
            start_in(g + 1, range(MAX_PATH))
    pltpu.make_async_copy(out_slice(0), obuf, sem_o).wait()


@functools.cache
def _sc_call():
    return functools.partial(
        pl.kernel,
        out_type=jax.ShapeDtypeStruct((N, N), jnp.float32),
        mesh=plsc.VectorSubcoreMesh(
            core_axis_name="c", subcore_axis_name="s",
            num_cores=_NC, num_subcores=_NS),
        compiler_params=pltpu.CompilerParams(needs_layout_passes=False),
        scratch_types=[
            pltpu.VMEM((_TBL_W,), jnp.float32),
        ] + [pltpu.VMEM((_G, N), jnp.int32) for _ in range(MAX_PATH)] + [
            pltpu.VMEM((_G, N), jnp.float32),
            pltpu.SemaphoreType.DMA,
            pltpu.SemaphoreType.DMA,
            pltpu.SemaphoreType.DMA,
        ],
    )(_sc_body)


def kernel(x, edge_attr, edge_paths, edge_vector):
    del x
    paths = jnp.transpose(edge_paths.astype(jnp.int32), (2, 0, 1))
    table = _make_table(edge_attr.astype(jnp.float32),
                        edge_vector.astype(jnp.float32))
    return _sc_call()(table, paths)
